# baseline (speedup 1.0000x reference)
import jax
import jax.numpy as jnp
from jax import lax
from jax.experimental import pallas as pl
from jax.experimental.pallas import tpu as pltpu

C = 256
T = 10000
THRESH = 0.6
H = 5000
NSTEPS = T // H


def _fused(d_ref, wg_ref, bg_ref, wc_ref, bc_ref, out_ref,
           g_acc, s1_acc, s2_acc, s3_acc, s4_acc):
    i = pl.program_id(0)
    d = d_ref[...]

    d2 = d * d
    s1 = jnp.sum(d, axis=0, keepdims=True)
    s2 = jnp.sum(d2, axis=0, keepdims=True)
    s3 = jnp.sum(d2 * d, axis=0, keepdims=True)
    s4 = jnp.sum(d2 * d2, axis=0, keepdims=True)
    db = d.astype(jnp.bfloat16)
    g = lax.dot_general(db, db, (((0,), (0,)), ((), ())),
                        preferred_element_type=jnp.float32)

    @pl.when(i == 0)
    def _():
        g_acc[...] = g
        s1_acc[...] = s1
        s2_acc[...] = s2
        s3_acc[...] = s3
        s4_acc[...] = s4

    @pl.when(i > 0)
    def _():
        g_acc[...] += g
        s1_acc[...] += s1
        s2_acc[...] += s2
        s3_acc[...] += s3
        s4_acc[...] += s4

    @pl.when(i == NSTEPS - 1)
    def _():
        inv_t = jnp.float32(1.0 / T)
        gt = g_acc[...]
        s1t, s2t = s1_acc[...], s2_acc[...]
        s3t, s4t = s3_acc[...], s4_acc[...]

        row = lax.broadcasted_iota(jnp.int32, (C, C), 0)
        colc = lax.broadcasted_iota(jnp.int32, (C, C), 1)
        on_diag = row == colc

        mu = s1t * inv_t
        mu_c = jnp.transpose(mu)

        normsq = jnp.maximum(s2t - (jnp.float32(T) * mu) * mu, 0.0)
        inv_norm = lax.rsqrt(jnp.maximum(normsq, jnp.float32(1e-12)))
        corr = ((gt - (jnp.float32(T) * mu_c) * mu)
                * inv_norm * jnp.transpose(inv_norm))
        corr = jnp.clip(corr, -1.0, 1.0)

        ac = jnp.abs(corr)
        mask = (ac >= jnp.float32(THRESH)) & (~on_diag)
        w = jnp.clip(ac, 1e-6, 0.99)
        a = jnp.where(mask, w, 0.0) + jnp.where(on_diag, 1.0, 0.0)

        m2 = s2t * inv_t - mu * mu
        m3 = s3t * inv_t - 3.0 * mu * (s2t * inv_t) + 2.0 * mu * mu * mu
        m4 = (s4t * inv_t - 4.0 * mu * (s3t * inv_t)
              + 6.0 * (mu * mu) * (s2t * inv_t) - 3.0 * (mu * mu) * (mu * mu))
        m2s = jnp.maximum(m2, jnp.float32(1e-12))
        inv_m2s = lax.rsqrt(m2s)
        skew = m3 * inv_m2s * inv_m2s * inv_m2s
        kurt = m4 * (inv_m2s * inv_m2s) * (inv_m2s * inv_m2s) - 3.0
        xt = jnp.concatenate([mu, m2, skew, kurt], axis=0)

        aggt = jnp.dot(xt, a, preferred_element_type=jnp.float32)
        ht = jnp.dot(wg_ref[...], aggt, preferred_element_type=jnp.float32)
        ht = jnp.maximum(ht + bg_ref[...], 0.0)
        get = jnp.sum(ht, axis=1, keepdims=True)
        ge = jnp.transpose(get)
        logits = jnp.dot(ge, wc_ref[...], preferred_element_type=jnp.float32)
        out_ref[...] = logits + bc_ref[...]


def kernel(data, W_gfc, b_gfc, W_cls, b_cls):
    dt = data.T
    out = pl.pallas_call(
        _fused,
        grid=(NSTEPS,),
        in_specs=[
            pl.BlockSpec((H, C), lambda i: (i, 0)),
            pl.BlockSpec((12, 4), lambda i: (0, 0)),
            pl.BlockSpec((12, 1), lambda i: (0, 0)),
            pl.BlockSpec((12, 2), lambda i: (0, 0)),
            pl.BlockSpec((1, 2), lambda i: (0, 0)),
        ],
        out_specs=pl.BlockSpec((1, 2), lambda i: (0, 0)),
        out_shape=jax.ShapeDtypeStruct((1, 2), jnp.float32),
        scratch_shapes=[
            pltpu.VMEM((C, C), jnp.float32),
            pltpu.VMEM((1, C), jnp.float32),
            pltpu.VMEM((1, C), jnp.float32),
            pltpu.VMEM((1, C), jnp.float32),
            pltpu.VMEM((1, C), jnp.float32),
        ],
    )(dt, W_gfc.T, b_gfc.reshape(-1, 1), W_cls, b_cls.reshape(1, -1))
    return out
